# baseline (speedup 1.0000x reference)
import dataclasses
import functools

import jax
import jax.numpy as jnp
from jax import lax
from jax.experimental import pallas as pl
from jax.experimental.pallas import tpu as pltpu
from jax.experimental.pallas import tpu_sc as plsc

K = 2
DD = 32
D = 64
N = 10000
M = 320000
ROUTIT = 3

NC = 2
NS = 16
NW = NC * NS
EPW = M // NW
CHUNK = 80
NCHUNK = EPW // CHUNK
ROWS_PER_SUB = 624
ROWS_TAIL = N - NS * ROWS_PER_SUB


def _normalize_halves(y):
    y0 = y[..., :DD]
    y1 = y[..., DD:]
    n0 = jnp.sqrt(jnp.sum(y0 * y0, axis=-1, keepdims=True))
    n1 = jnp.sqrt(jnp.sum(y1 * y1, axis=-1, keepdims=True))
    y0 = y0 / jnp.maximum(n0, 1e-12)
    y1 = y1 / jnp.maximum(n1, 1e-12)
    return jnp.concatenate([y0, y1], axis=-1)


def _tc_linear_body(x_ref, w_ref, b_ref, o_ref):
    y = lax.dot_general(
        x_ref[...], w_ref[...], (((1,), (1,)), ((), ())),
        preferred_element_type=jnp.float32,
        precision=lax.Precision.HIGHEST,
    )
    y = y + b_ref[...]
    o_ref[...] = _normalize_halves(y)


def _tc_linear(x, W, b):
    return pl.pallas_call(
        _tc_linear_body,
        out_shape=jax.ShapeDtypeStruct((N, D), jnp.float32),
    )(x, W, b.reshape(1, D))


def _tc_combine_body(c_ref, d_ref, o_ref):
    y = c_ref[...] + d_ref[0] + d_ref[1]
    o_ref[...] = _normalize_halves(y)


def _tc_combine(c, delta):
    return pl.pallas_call(
        _tc_combine_body,
        out_shape=jax.ShapeDtypeStruct((N, D), jnp.float32),
    )(c, delta)


def _sc_route_body(xn_hbm, c_hbm, src_hbm, trg_hbm, zeros_hbm, out_hbm,
                   src_v0, trg_v0, src_v1, trg_v1, src_v2, trg_v2,
                   z_v0, cg_v0, z_v1, cg_v1, z_v2, cg_v2,
                   w_v0, w_v1, w_v2, pb_v, p0_v, p1_v,
                   sem0, sem1, sem2, semw0, semw1, semw2, acc_sh):
    cidx = lax.axis_index("c")
    sid = lax.axis_index("s")
    wid = sid * NC + cidx

    src_b = (src_v0, src_v1, src_v2)
    trg_b = (trg_v0, trg_v1, trg_v2)
    z_b = (z_v0, z_v1, z_v2)
    cg_b = (cg_v0, cg_v1, cg_v2)
    w_b = (w_v0, w_v1, w_v2)
    sem_b = (sem0, sem1, sem2)
    semw_b = (semw0, semw1, semw2)

    pltpu.sync_copy(zeros_hbm.at[pl.ds(sid * ROWS_PER_SUB, ROWS_PER_SUB)],
                    acc_sh.at[pl.ds(sid * ROWS_PER_SUB, ROWS_PER_SUB)])

    @pl.when(sid == 0)
    def _zero_tail():
        pltpu.sync_copy(zeros_hbm.at[pl.ds(NS * ROWS_PER_SUB, ROWS_TAIL)],
                        acc_sh.at[pl.ds(NS * ROWS_PER_SUB, ROWS_TAIL)])

    plsc.subcore_barrier()

    iot = lax.iota(jnp.int32, 16)
    ebase = wid * EPW

    def drain_scatter(b):
        pltpu.make_async_copy(w_b[b], acc_sh.at[trg_b[b]], semw_b[b]).wait()

    def start_fetch(ci, b):
        pltpu.sync_copy(src_hbm.at[pl.ds(ebase + ci * CHUNK, CHUNK)], src_b[b])
        pltpu.sync_copy(trg_hbm.at[pl.ds(ebase + ci * CHUNK, CHUNK)], trg_b[b])
        pltpu.async_copy(xn_hbm.at[src_b[b]], z_b[b], sem_b[b])
        pltpu.async_copy(c_hbm.at[trg_b[b]], cg_b[b], sem_b[b])

    def finish_chunk(b):
        pltpu.make_async_copy(xn_hbm.at[src_b[b]], z_b[b], sem_b[b]).wait()
        pltpu.make_async_copy(c_hbm.at[trg_b[b]], cg_b[b], sem_b[b]).wait()
        z_v = z_b[b]
        cg_v = cg_b[b]
        w_v = w_b[b]

        for g in range(CHUNK // 16):
            for e in range(16):
                row = g * 16 + e
                part = (z_v[row, pl.ds(2 * 16, 16)] * cg_v[row, pl.ds(2 * 16, 16)]
                        + z_v[row, pl.ds(3 * 16, 16)] * cg_v[row, pl.ds(3 * 16, 16)]
                        - z_v[row, pl.ds(0, 16)] * cg_v[row, pl.ds(0, 16)]
                        - z_v[row, pl.ds(16, 16)] * cg_v[row, pl.ds(16, 16)])
                pb_v[e, :] = part
            dsum = jnp.zeros((16,), jnp.float32)
            for l in range(16):
                col = plsc.load_gather(pb_v, [iot, jnp.full((16,), l, jnp.int32)])
                dsum = dsum + col
            p1 = 1.0 / (1.0 + jnp.exp(-dsum))
            p0_v[...] = 1.0 - p1
            p1_v[...] = p1
            p0vec = p0_v[...]
            p1vec = p1_v[...]
            for e in range(16):
                row = g * 16 + e
                b0 = jnp.full((16,), p0vec[e], jnp.float32)
                b1 = jnp.full((16,), p1vec[e], jnp.float32)
                w_v[row, pl.ds(0, 16)] = z_v[row, pl.ds(0, 16)] * b0
                w_v[row, pl.ds(16, 16)] = z_v[row, pl.ds(16, 16)] * b0
                w_v[row, pl.ds(32, 16)] = z_v[row, pl.ds(32, 16)] * b1
                w_v[row, pl.ds(48, 16)] = z_v[row, pl.ds(48, 16)] * b1

        pltpu.async_copy(w_v, acc_sh.at[trg_b[b]], semw_b[b], add=True)

    start_fetch(0, 0)

    @pl.loop(0, NCHUNK - 2, step=3)
    def _chunk3(ci0):
        @pl.when(ci0 > 0)
        def _d1():
            drain_scatter(1)
        start_fetch(ci0 + 1, 1)
        finish_chunk(0)

        @pl.when(ci0 > 0)
        def _d2():
            drain_scatter(2)
        start_fetch(ci0 + 2, 2)
        finish_chunk(1)

        drain_scatter(0)
        start_fetch(ci0 + 3, 0)
        finish_chunk(2)

    drain_scatter(1)
    start_fetch(NCHUNK - 1, 1)
    finish_chunk(0)
    finish_chunk(1)

    drain_scatter(2)
    drain_scatter(0)
    drain_scatter(1)

    plsc.subcore_barrier()
    pltpu.sync_copy(acc_sh.at[pl.ds(sid * ROWS_PER_SUB, ROWS_PER_SUB)],
                    out_hbm.at[cidx, pl.ds(sid * ROWS_PER_SUB, ROWS_PER_SUB)])

    @pl.when(sid == 0)
    def _out_tail():
        pltpu.sync_copy(acc_sh.at[pl.ds(NS * ROWS_PER_SUB, ROWS_TAIL)],
                        out_hbm.at[cidx, pl.ds(NS * ROWS_PER_SUB, ROWS_TAIL)])


def _sc_route(xn, c, src, trg, zeros):
    mesh = plsc.VectorSubcoreMesh(core_axis_name="c", subcore_axis_name="s",
                                  num_cores=NC, num_subcores=NS)
    cp = pltpu.CompilerParams(use_tc_tiling_on_sc=False,
                              needs_layout_passes=False)
    f = pl.kernel(
        _sc_route_body,
        out_type=jax.ShapeDtypeStruct((NC, N, D), jnp.float32),
        mesh=mesh,
        scratch_types=(
            [pltpu.VMEM((CHUNK,), jnp.int32)] * 6
            + [pltpu.VMEM((CHUNK, D), jnp.float32)] * 6
            + [pltpu.VMEM((CHUNK, D), jnp.float32)] * 3
            + [pltpu.VMEM((16, 16), jnp.float32),
               pltpu.VMEM((16,), jnp.float32),
               pltpu.VMEM((16,), jnp.float32)]
            + [pltpu.SemaphoreType.DMA] * 6
            + [pltpu.VMEM_SHARED((N, D), jnp.float32)]
        ),
        compiler_params=cp,
    )
    return f(xn, c, src, trg, zeros)


def kernel(x, src_trg, W, b):
    src = src_trg[0].astype(jnp.int32)
    trg = src_trg[1].astype(jnp.int32)
    xn = _tc_linear(x, W, b)
    zeros = jnp.zeros((N, D), jnp.float32)
    c = xn
    for _ in range(ROUTIT):
        delta = _sc_route(xn, c, src, trg, zeros)
        c = _tc_combine(c, delta)
    return c


# speedup vs baseline: 1.5502x; 1.5502x over previous
import dataclasses
import functools

import jax
import jax.numpy as jnp
from jax import lax
from jax.experimental import pallas as pl
from jax.experimental.pallas import tpu as pltpu
from jax.experimental.pallas import tpu_sc as plsc

K = 2
DD = 32
D = 64
N = 10000
M = 320000
ROUTIT = 3

NC = 2
NS = 16
NW = NC * NS
EPW = M // NW
CHUNK = 80
NCHUNK = EPW // CHUNK
ROWS_PER_SUB = 624
ROWS_TAIL = N - NS * ROWS_PER_SUB


def _normalize_halves(y):
    y0 = y[..., :DD]
    y1 = y[..., DD:]
    n0 = jnp.sqrt(jnp.sum(y0 * y0, axis=-1, keepdims=True))
    n1 = jnp.sqrt(jnp.sum(y1 * y1, axis=-1, keepdims=True))
    y0 = y0 / jnp.maximum(n0, 1e-12)
    y1 = y1 / jnp.maximum(n1, 1e-12)
    return jnp.concatenate([y0, y1], axis=-1)


def _tc_linear_body(x_ref, w_ref, b_ref, o_ref):
    y = lax.dot_general(
        x_ref[...], w_ref[...], (((1,), (1,)), ((), ())),
        preferred_element_type=jnp.float32,
        precision=lax.Precision.HIGHEST,
    )
    y = y + b_ref[...]
    o_ref[...] = _normalize_halves(y)


def _tc_linear(x, W, b):
    return pl.pallas_call(
        _tc_linear_body,
        out_shape=jax.ShapeDtypeStruct((N, D), jnp.float32),
    )(x, W, b.reshape(1, D))


def _tc_combine_body(c_ref, d_ref, o_ref):
    y = c_ref[...] + d_ref[0] + d_ref[1]
    o_ref[...] = _normalize_halves(y)


def _tc_combine(c, delta):
    return pl.pallas_call(
        _tc_combine_body,
        out_shape=jax.ShapeDtypeStruct((N, D), jnp.float32),
    )(c, delta)


def _sc_route_body(xn_hbm, c_hbm, src_hbm, trg_hbm, zeros_hbm, out_hbm,
                   src_v0, trg_v0, src_v1, trg_v1, src_v2, trg_v2,
                   z_v0, cg_v0, z_v1, cg_v1, z_v2, cg_v2,
                   w_v0, w_v1, w_v2, pb_v, p0_v, p1_v,
                   sem0, sem1, sem2, semw0, semw1, semw2, acc_sh):
    cidx = lax.axis_index("c")
    sid = lax.axis_index("s")
    wid = sid * NC + cidx

    src_b = (src_v0, src_v1, src_v2)
    trg_b = (trg_v0, trg_v1, trg_v2)
    z_b = (z_v0, z_v1, z_v2)
    cg_b = (cg_v0, cg_v1, cg_v2)
    w_b = (w_v0, w_v1, w_v2)
    sem_b = (sem0, sem1, sem2)
    semw_b = (semw0, semw1, semw2)

    pltpu.sync_copy(zeros_hbm.at[pl.ds(sid * ROWS_PER_SUB, ROWS_PER_SUB)],
                    acc_sh.at[pl.ds(sid * ROWS_PER_SUB, ROWS_PER_SUB)])

    @pl.when(sid == 0)
    def _zero_tail():
        pltpu.sync_copy(zeros_hbm.at[pl.ds(NS * ROWS_PER_SUB, ROWS_TAIL)],
                        acc_sh.at[pl.ds(NS * ROWS_PER_SUB, ROWS_TAIL)])

    plsc.subcore_barrier()

    iot = lax.iota(jnp.int32, 16)
    ebase = wid * EPW

    def drain_scatter(b):
        pltpu.make_async_copy(w_b[b], acc_sh.at[trg_b[b]], semw_b[b]).wait()

    def start_fetch(ci, b):
        pltpu.sync_copy(src_hbm.at[pl.ds(ebase + ci * CHUNK, CHUNK)], src_b[b])
        pltpu.sync_copy(trg_hbm.at[pl.ds(ebase + ci * CHUNK, CHUNK)], trg_b[b])
        pltpu.async_copy(xn_hbm.at[src_b[b]], z_b[b], sem_b[b])
        pltpu.async_copy(c_hbm.at[trg_b[b]], cg_b[b], sem_b[b])

    def finish_chunk(b):
        pltpu.make_async_copy(xn_hbm.at[src_b[b]], z_b[b], sem_b[b]).wait()
        pltpu.make_async_copy(c_hbm.at[trg_b[b]], cg_b[b], sem_b[b]).wait()
        z_v = z_b[b]
        cg_v = cg_b[b]
        w_v = w_b[b]

        @pl.loop(0, CHUNK // 16)
        def _group(g):
            for e in range(16):
                row = g * 16 + e
                part = (z_v[row, pl.ds(2 * 16, 16)] * cg_v[row, pl.ds(2 * 16, 16)]
                        + z_v[row, pl.ds(3 * 16, 16)] * cg_v[row, pl.ds(3 * 16, 16)]
                        - z_v[row, pl.ds(0, 16)] * cg_v[row, pl.ds(0, 16)]
                        - z_v[row, pl.ds(16, 16)] * cg_v[row, pl.ds(16, 16)])
                pb_v[e, :] = part
            dsum = jnp.zeros((16,), jnp.float32)
            for l in range(16):
                col = plsc.load_gather(pb_v, [iot, jnp.full((16,), l, jnp.int32)])
                dsum = dsum + col
            p1 = 1.0 / (1.0 + jnp.exp(-dsum))
            p0_v[...] = 1.0 - p1
            p1_v[...] = p1
            p0vec = p0_v[...]
            p1vec = p1_v[...]
            for e in range(16):
                row = g * 16 + e
                b0 = jnp.full((16,), p0vec[e], jnp.float32)
                b1 = jnp.full((16,), p1vec[e], jnp.float32)
                w_v[row, pl.ds(0, 16)] = z_v[row, pl.ds(0, 16)] * b0
                w_v[row, pl.ds(16, 16)] = z_v[row, pl.ds(16, 16)] * b0
                w_v[row, pl.ds(32, 16)] = z_v[row, pl.ds(32, 16)] * b1
                w_v[row, pl.ds(48, 16)] = z_v[row, pl.ds(48, 16)] * b1

        pltpu.async_copy(w_v, acc_sh.at[trg_b[b]], semw_b[b], add=True)

    start_fetch(0, 0)

    @pl.loop(0, NCHUNK - 2, step=3)
    def _chunk3(ci0):
        @pl.when(ci0 > 0)
        def _d1():
            drain_scatter(1)
        start_fetch(ci0 + 1, 1)
        finish_chunk(0)

        @pl.when(ci0 > 0)
        def _d2():
            drain_scatter(2)
        start_fetch(ci0 + 2, 2)
        finish_chunk(1)

        drain_scatter(0)
        start_fetch(ci0 + 3, 0)
        finish_chunk(2)

    drain_scatter(1)
    start_fetch(NCHUNK - 1, 1)
    finish_chunk(0)
    finish_chunk(1)

    drain_scatter(2)
    drain_scatter(0)
    drain_scatter(1)

    plsc.subcore_barrier()
    pltpu.sync_copy(acc_sh.at[pl.ds(sid * ROWS_PER_SUB, ROWS_PER_SUB)],
                    out_hbm.at[cidx, pl.ds(sid * ROWS_PER_SUB, ROWS_PER_SUB)])

    @pl.when(sid == 0)
    def _out_tail():
        pltpu.sync_copy(acc_sh.at[pl.ds(NS * ROWS_PER_SUB, ROWS_TAIL)],
                        out_hbm.at[cidx, pl.ds(NS * ROWS_PER_SUB, ROWS_TAIL)])


def _sc_route(xn, c, src, trg, zeros):
    mesh = plsc.VectorSubcoreMesh(core_axis_name="c", subcore_axis_name="s",
                                  num_cores=NC, num_subcores=NS)
    cp = pltpu.CompilerParams(use_tc_tiling_on_sc=False,
                              needs_layout_passes=False)
    f = pl.kernel(
        _sc_route_body,
        out_type=jax.ShapeDtypeStruct((NC, N, D), jnp.float32),
        mesh=mesh,
        scratch_types=(
            [pltpu.VMEM((CHUNK,), jnp.int32)] * 6
            + [pltpu.VMEM((CHUNK, D), jnp.float32)] * 6
            + [pltpu.VMEM((CHUNK, D), jnp.float32)] * 3
            + [pltpu.VMEM((16, 16), jnp.float32),
               pltpu.VMEM((16,), jnp.float32),
               pltpu.VMEM((16,), jnp.float32)]
            + [pltpu.SemaphoreType.DMA] * 6
            + [pltpu.VMEM_SHARED((N, D), jnp.float32)]
        ),
        compiler_params=cp,
    )
    return f(xn, c, src, trg, zeros)


def kernel(x, src_trg, W, b):
    src = src_trg[0].astype(jnp.int32)
    trg = src_trg[1].astype(jnp.int32)
    xn = _tc_linear(x, W, b)
    zeros = jnp.zeros((N, D), jnp.float32)
    c = xn
    for _ in range(ROUTIT):
        delta = _sc_route(xn, c, src, trg, zeros)
        c = _tc_combine(c, delta)
    return c


# speedup vs baseline: 1.5595x; 1.0060x over previous
import dataclasses
import functools

import jax
import jax.numpy as jnp
from jax import lax
from jax.experimental import pallas as pl
from jax.experimental.pallas import tpu as pltpu
from jax.experimental.pallas import tpu_sc as plsc

K = 2
DD = 32
D = 64
N = 10000
M = 320000
ROUTIT = 3

NC = 2
NS = 16
NW = NC * NS
EPW = M // NW
CHUNK = 80
NCHUNK = EPW // CHUNK
ROWS_PER_SUB = 624
ROWS_TAIL = N - NS * ROWS_PER_SUB


def _normalize_halves(y):
    y0 = y[..., :DD]
    y1 = y[..., DD:]
    n0 = jnp.sqrt(jnp.sum(y0 * y0, axis=-1, keepdims=True))
    n1 = jnp.sqrt(jnp.sum(y1 * y1, axis=-1, keepdims=True))
    y0 = y0 / jnp.maximum(n0, 1e-12)
    y1 = y1 / jnp.maximum(n1, 1e-12)
    return jnp.concatenate([y0, y1], axis=-1)


def _tc_linear_body(x_ref, w_ref, b_ref, o_ref):
    y = lax.dot_general(
        x_ref[...], w_ref[...], (((1,), (1,)), ((), ())),
        preferred_element_type=jnp.float32,
        precision=lax.Precision.HIGHEST,
    )
    y = y + b_ref[...]
    o_ref[...] = _normalize_halves(y)


def _tc_linear(x, W, b):
    return pl.pallas_call(
        _tc_linear_body,
        out_shape=jax.ShapeDtypeStruct((N, D), jnp.float32),
    )(x, W, b.reshape(1, D))


def _tc_combine_body(c_ref, d_ref, o_ref):
    y = c_ref[...] + d_ref[0] + d_ref[1]
    o_ref[...] = _normalize_halves(y)


def _tc_combine(c, delta):
    return pl.pallas_call(
        _tc_combine_body,
        out_shape=jax.ShapeDtypeStruct((N, D), jnp.float32),
    )(c, delta)


def _sc_route_body(xn_hbm, c_hbm, src_hbm, trg_hbm, zeros_hbm, out_hbm,
                   src_v0, trg_v0, src_v1, trg_v1, src_v2, trg_v2,
                   z_v0, cg_v0, z_v1, cg_v1, z_v2, cg_v2,
                   w_v0, w_v1, w_v2, pb_v, p0_v, p1_v,
                   sem0, sem1, sem2, semw0, semw1, semw2, acc_sh):
    cidx = lax.axis_index("c")
    sid = lax.axis_index("s")
    wid = sid * NC + cidx

    src_b = (src_v0, src_v1, src_v2)
    trg_b = (trg_v0, trg_v1, trg_v2)
    z_b = (z_v0, z_v1, z_v2)
    cg_b = (cg_v0, cg_v1, cg_v2)
    w_b = (w_v0, w_v1, w_v2)
    sem_b = (sem0, sem1, sem2)
    semw_b = (semw0, semw1, semw2)

    pltpu.sync_copy(zeros_hbm.at[pl.ds(sid * ROWS_PER_SUB, ROWS_PER_SUB)],
                    acc_sh.at[pl.ds(sid * ROWS_PER_SUB, ROWS_PER_SUB)])

    @pl.when(sid == 0)
    def _zero_tail():
        pltpu.sync_copy(zeros_hbm.at[pl.ds(NS * ROWS_PER_SUB, ROWS_TAIL)],
                        acc_sh.at[pl.ds(NS * ROWS_PER_SUB, ROWS_TAIL)])

    plsc.subcore_barrier()

    iot = lax.iota(jnp.int32, 16)
    ebase = wid * EPW

    def drain_scatter(b):
        pltpu.make_async_copy(w_b[b], acc_sh.at[trg_b[b]], semw_b[b]).wait()

    def start_fetch(ci, b):
        pltpu.sync_copy(src_hbm.at[pl.ds(ebase + ci * CHUNK, CHUNK)], src_b[b])
        pltpu.sync_copy(trg_hbm.at[pl.ds(ebase + ci * CHUNK, CHUNK)], trg_b[b])
        pltpu.async_copy(xn_hbm.at[src_b[b]], z_b[b], sem_b[b])
        pltpu.async_copy(c_hbm.at[trg_b[b]], cg_b[b], sem_b[b])

    def finish_chunk(b):
        pltpu.make_async_copy(xn_hbm.at[src_b[b]], z_b[b], sem_b[b]).wait()
        pltpu.make_async_copy(c_hbm.at[trg_b[b]], cg_b[b], sem_b[b]).wait()
        z_v = z_b[b]
        cg_v = cg_b[b]
        w_v = w_b[b]

        @pl.loop(0, CHUNK // 16)
        def _group(g):
            for e in range(16):
                row = g * 16 + e
                part = (z_v[row, pl.ds(2 * 16, 16)] * cg_v[row, pl.ds(2 * 16, 16)]
                        + z_v[row, pl.ds(3 * 16, 16)] * cg_v[row, pl.ds(3 * 16, 16)]
                        - z_v[row, pl.ds(0, 16)] * cg_v[row, pl.ds(0, 16)]
                        - z_v[row, pl.ds(16, 16)] * cg_v[row, pl.ds(16, 16)])
                pb_v[e, :] = part
            dsum = jnp.zeros((16,), jnp.float32)
            for l in range(16):
                col = plsc.load_gather(pb_v, [iot, jnp.full((16,), l, jnp.int32)])
                dsum = dsum + col
            p1 = 1.0 / (1.0 + jnp.exp(-dsum))
            for e in range(16):
                row = g * 16 + e
                b1 = jnp.full((16,), p1[e], jnp.float32)
                b0 = 1.0 - b1
                w_v[row, pl.ds(0, 16)] = z_v[row, pl.ds(0, 16)] * b0
                w_v[row, pl.ds(16, 16)] = z_v[row, pl.ds(16, 16)] * b0
                w_v[row, pl.ds(32, 16)] = z_v[row, pl.ds(32, 16)] * b1
                w_v[row, pl.ds(48, 16)] = z_v[row, pl.ds(48, 16)] * b1

        pltpu.async_copy(w_v, acc_sh.at[trg_b[b]], semw_b[b], add=True)

    start_fetch(0, 0)

    @pl.loop(0, NCHUNK - 2, step=3)
    def _chunk3(ci0):
        @pl.when(ci0 > 0)
        def _d1():
            drain_scatter(1)
        start_fetch(ci0 + 1, 1)
        finish_chunk(0)

        @pl.when(ci0 > 0)
        def _d2():
            drain_scatter(2)
        start_fetch(ci0 + 2, 2)
        finish_chunk(1)

        drain_scatter(0)
        start_fetch(ci0 + 3, 0)
        finish_chunk(2)

    drain_scatter(1)
    start_fetch(NCHUNK - 1, 1)
    finish_chunk(0)
    finish_chunk(1)

    drain_scatter(2)
    drain_scatter(0)
    drain_scatter(1)

    plsc.subcore_barrier()
    pltpu.sync_copy(acc_sh.at[pl.ds(sid * ROWS_PER_SUB, ROWS_PER_SUB)],
                    out_hbm.at[cidx, pl.ds(sid * ROWS_PER_SUB, ROWS_PER_SUB)])

    @pl.when(sid == 0)
    def _out_tail():
        pltpu.sync_copy(acc_sh.at[pl.ds(NS * ROWS_PER_SUB, ROWS_TAIL)],
                        out_hbm.at[cidx, pl.ds(NS * ROWS_PER_SUB, ROWS_TAIL)])


def _sc_route(xn, c, src, trg, zeros):
    mesh = plsc.VectorSubcoreMesh(core_axis_name="c", subcore_axis_name="s",
                                  num_cores=NC, num_subcores=NS)
    cp = pltpu.CompilerParams(use_tc_tiling_on_sc=False,
                              needs_layout_passes=False)
    f = pl.kernel(
        _sc_route_body,
        out_type=jax.ShapeDtypeStruct((NC, N, D), jnp.float32),
        mesh=mesh,
        scratch_types=(
            [pltpu.VMEM((CHUNK,), jnp.int32)] * 6
            + [pltpu.VMEM((CHUNK, D), jnp.float32)] * 6
            + [pltpu.VMEM((CHUNK, D), jnp.float32)] * 3
            + [pltpu.VMEM((16, 16), jnp.float32),
               pltpu.VMEM((16,), jnp.float32),
               pltpu.VMEM((16,), jnp.float32)]
            + [pltpu.SemaphoreType.DMA] * 6
            + [pltpu.VMEM_SHARED((N, D), jnp.float32)]
        ),
        compiler_params=cp,
    )
    return f(xn, c, src, trg, zeros)


def kernel(x, src_trg, W, b):
    src = src_trg[0].astype(jnp.int32)
    trg = src_trg[1].astype(jnp.int32)
    xn = _tc_linear(x, W, b)
    zeros = jnp.zeros((N, D), jnp.float32)
    c = xn
    for _ in range(ROUTIT):
        delta = _sc_route(xn, c, src, trg, zeros)
        c = _tc_combine(c, delta)
    return c


# speedup vs baseline: 2.2815x; 1.4630x over previous
import dataclasses
import functools

import jax
import jax.numpy as jnp
from jax import lax
from jax.experimental import pallas as pl
from jax.experimental.pallas import tpu as pltpu
from jax.experimental.pallas import tpu_sc as plsc

K = 2
DD = 32
D = 64
N = 10000
M = 320000
ROUTIT = 3

NC = 2
NS = 16
NW = NC * NS
EPW = M // NW
CHUNK = 80
NCHUNK = EPW // CHUNK
ROWS_PER_SUB = 624
ROWS_TAIL = N - NS * ROWS_PER_SUB


def _normalize_halves(y):
    y0 = y[..., :DD]
    y1 = y[..., DD:]
    n0 = jnp.sqrt(jnp.sum(y0 * y0, axis=-1, keepdims=True))
    n1 = jnp.sqrt(jnp.sum(y1 * y1, axis=-1, keepdims=True))
    y0 = y0 / jnp.maximum(n0, 1e-12)
    y1 = y1 / jnp.maximum(n1, 1e-12)
    return jnp.concatenate([y0, y1], axis=-1)


def _tc_linear_body(x_ref, w_ref, b_ref, o_ref):
    y = lax.dot_general(
        x_ref[...], w_ref[...], (((1,), (1,)), ((), ())),
        preferred_element_type=jnp.float32,
        precision=lax.Precision.HIGHEST,
    )
    y = y + b_ref[...]
    o_ref[...] = _normalize_halves(y)


def _tc_linear(x, W, b):
    return pl.pallas_call(
        _tc_linear_body,
        out_shape=jax.ShapeDtypeStruct((N, D), jnp.float32),
    )(x, W, b.reshape(1, D))


def _tc_combine_body(c_ref, d_ref, o_ref):
    y = c_ref[...] + d_ref[0] + d_ref[1]
    o_ref[...] = _normalize_halves(y)


def _tc_combine(c, delta):
    return pl.pallas_call(
        _tc_combine_body,
        out_shape=jax.ShapeDtypeStruct((N, D), jnp.float32),
    )(c, delta)


def _sc_route_body(xn_hbm, c_hbm, src_hbm, trg_hbm, zeros_hbm, out_hbm,
                   src_v0, trg_v0, src_v1, trg_v1, src_v2, trg_v2,
                   ts_v0, ts_v1, ts_v2,
                   z_v0, cg_v0, z_v1, cg_v1, z_v2, cg_v2,
                   w_v0, w_v1, w_v2, pb_v, p0_v, p1_v,
                   semi0, semi1, semi2, sem0, sem1, sem2,
                   semw0, semw1, semw2, acc_sh):
    cidx = lax.axis_index("c")
    sid = lax.axis_index("s")
    wid = sid * NC + cidx

    src_b = (src_v0, src_v1, src_v2)
    trg_b = (trg_v0, trg_v1, trg_v2)
    ts_b = (ts_v0, ts_v1, ts_v2)
    z_b = (z_v0, z_v1, z_v2)
    cg_b = (cg_v0, cg_v1, cg_v2)
    w_b = (w_v0, w_v1, w_v2)
    semi_b = (semi0, semi1, semi2)
    sem_b = (sem0, sem1, sem2)
    semw_b = (semw0, semw1, semw2)

    pltpu.sync_copy(zeros_hbm.at[pl.ds(sid * ROWS_PER_SUB, ROWS_PER_SUB)],
                    acc_sh.at[pl.ds(sid * ROWS_PER_SUB, ROWS_PER_SUB)])

    @pl.when(sid == 0)
    def _zero_tail():
        pltpu.sync_copy(zeros_hbm.at[pl.ds(NS * ROWS_PER_SUB, ROWS_TAIL)],
                        acc_sh.at[pl.ds(NS * ROWS_PER_SUB, ROWS_TAIL)])

    plsc.subcore_barrier()

    iot = lax.iota(jnp.int32, 16)
    ebase = wid * EPW

    def drain_scatter(b):
        pltpu.make_async_copy(w_b[b], acc_sh.at[ts_b[b]], semw_b[b]).wait()

    def idx_fetch(ci, b):
        pltpu.async_copy(src_hbm.at[pl.ds(ebase + ci * CHUNK, CHUNK)],
                         src_b[b], semi_b[b])
        pltpu.async_copy(trg_hbm.at[pl.ds(ebase + ci * CHUNK, CHUNK)],
                         trg_b[b], semi_b[b])

    def row_fetch(ci, b):
        pltpu.make_async_copy(src_hbm.at[pl.ds(ebase + ci * CHUNK, CHUNK)],
                              src_b[b], semi_b[b]).wait()
        pltpu.make_async_copy(trg_hbm.at[pl.ds(ebase + ci * CHUNK, CHUNK)],
                              trg_b[b], semi_b[b]).wait()
        pltpu.async_copy(xn_hbm.at[src_b[b]], z_b[b], sem_b[b])
        pltpu.async_copy(c_hbm.at[trg_b[b]], cg_b[b], sem_b[b])

    def finish_chunk(b):
        pltpu.make_async_copy(xn_hbm.at[src_b[b]], z_b[b], sem_b[b]).wait()
        pltpu.make_async_copy(c_hbm.at[trg_b[b]], cg_b[b], sem_b[b]).wait()
        for q in range(CHUNK // 16):
            ts_b[b][pl.ds(q * 16, 16)] = trg_b[b][pl.ds(q * 16, 16)]
        z_v = z_b[b]
        cg_v = cg_b[b]
        w_v = w_b[b]

        @pl.loop(0, CHUNK // 16)
        def _group(g):
            for e in range(16):
                row = g * 16 + e
                part = (z_v[row, pl.ds(2 * 16, 16)] * cg_v[row, pl.ds(2 * 16, 16)]
                        + z_v[row, pl.ds(3 * 16, 16)] * cg_v[row, pl.ds(3 * 16, 16)]
                        - z_v[row, pl.ds(0, 16)] * cg_v[row, pl.ds(0, 16)]
                        - z_v[row, pl.ds(16, 16)] * cg_v[row, pl.ds(16, 16)])
                pb_v[e, :] = part
            dsum = jnp.zeros((16,), jnp.float32)
            for l in range(16):
                col = plsc.load_gather(pb_v, [iot, jnp.full((16,), l, jnp.int32)])
                dsum = dsum + col
            p1 = 1.0 / (1.0 + jnp.exp(-dsum))
            for e in range(16):
                row = g * 16 + e
                b1 = jnp.full((16,), p1[e], jnp.float32)
                b0 = 1.0 - b1
                w_v[row, pl.ds(0, 16)] = z_v[row, pl.ds(0, 16)] * b0
                w_v[row, pl.ds(16, 16)] = z_v[row, pl.ds(16, 16)] * b0
                w_v[row, pl.ds(32, 16)] = z_v[row, pl.ds(32, 16)] * b1
                w_v[row, pl.ds(48, 16)] = z_v[row, pl.ds(48, 16)] * b1

        pltpu.async_copy(w_v, acc_sh.at[ts_b[b]], semw_b[b], add=True)

    idx_fetch(0, 0)
    idx_fetch(1, 1)
    row_fetch(0, 0)

    @pl.loop(0, NCHUNK - 2, step=3)
    def _chunk3(ci0):
        idx_fetch(ci0 + 2, 2)
        row_fetch(ci0 + 1, 1)

        @pl.when(ci0 > 0)
        def _d0():
            drain_scatter(0)
        finish_chunk(0)

        idx_fetch(ci0 + 3, 0)
        row_fetch(ci0 + 2, 2)

        @pl.when(ci0 > 0)
        def _d1():
            drain_scatter(1)
        finish_chunk(1)

        idx_fetch(ci0 + 4, 1)
        row_fetch(ci0 + 3, 0)

        @pl.when(ci0 > 0)
        def _d2():
            drain_scatter(2)
        finish_chunk(2)

    row_fetch(NCHUNK - 1, 1)
    drain_scatter(0)
    finish_chunk(0)
    drain_scatter(1)
    finish_chunk(1)

    drain_scatter(2)
    drain_scatter(0)
    drain_scatter(1)

    plsc.subcore_barrier()
    pltpu.sync_copy(acc_sh.at[pl.ds(sid * ROWS_PER_SUB, ROWS_PER_SUB)],
                    out_hbm.at[cidx, pl.ds(sid * ROWS_PER_SUB, ROWS_PER_SUB)])

    @pl.when(sid == 0)
    def _out_tail():
        pltpu.sync_copy(acc_sh.at[pl.ds(NS * ROWS_PER_SUB, ROWS_TAIL)],
                        out_hbm.at[cidx, pl.ds(NS * ROWS_PER_SUB, ROWS_TAIL)])


def _sc_route(xn, c, src, trg, zeros):
    mesh = plsc.VectorSubcoreMesh(core_axis_name="c", subcore_axis_name="s",
                                  num_cores=NC, num_subcores=NS)
    cp = pltpu.CompilerParams(use_tc_tiling_on_sc=False,
                              needs_layout_passes=False)
    f = pl.kernel(
        _sc_route_body,
        out_type=jax.ShapeDtypeStruct((NC, N, D), jnp.float32),
        mesh=mesh,
        scratch_types=(
            [pltpu.VMEM((CHUNK,), jnp.int32)] * 9
            + [pltpu.VMEM((CHUNK, D), jnp.float32)] * 6
            + [pltpu.VMEM((CHUNK, D), jnp.float32)] * 3
            + [pltpu.VMEM((16, 16), jnp.float32),
               pltpu.VMEM((16,), jnp.float32),
               pltpu.VMEM((16,), jnp.float32)]
            + [pltpu.SemaphoreType.DMA] * 9
            + [pltpu.VMEM_SHARED((N, D), jnp.float32)]
        ),
        compiler_params=cp,
    )
    return f(xn, c, src, trg, zeros)


def kernel(x, src_trg, W, b):
    src = src_trg[0].astype(jnp.int32)
    trg = src_trg[1].astype(jnp.int32)
    xn = _tc_linear(x, W, b)
    zeros = jnp.zeros((N, D), jnp.float32)
    c = xn
    for _ in range(ROUTIT):
        delta = _sc_route(xn, c, src, trg, zeros)
        c = _tc_combine(c, delta)
    return c
